# baseline (speedup 1.0000x reference)
import functools

import jax
import jax.numpy as jnp
from jax import lax
from jax.experimental import pallas as pl
from jax.experimental.pallas import tpu as pltpu
from jax.experimental.pallas import tpu_sc as plsc

K = 100000
DIM = 128
B = 16384

_INFO = plsc.get_sparse_core_info()
_NC, _NS = _INFO.num_cores, _INFO.num_subcores
_NW = _NC * _NS

_CH = 256
_VCH = B // _CH
_NCHUNK = _VCH + (K - B + _CH - 1) // _CH
_NITER = (_NCHUNK + _NW - 1) // _NW


@functools.partial(
    pl.kernel,
    mesh=plsc.VectorSubcoreMesh(core_axis_name="c", subcore_axis_name="s"),
    out_type=jax.ShapeDtypeStruct((K, DIM), jnp.float32),
    scratch_types=[
        pltpu.VMEM((_CH, DIM), jnp.float32),
        pltpu.VMEM((_CH, DIM), jnp.float32),
        pltpu.SemaphoreType.DMA,
        pltpu.SemaphoreType.DMA,
        pltpu.SemaphoreType.DMA,
        pltpu.SemaphoreType.DMA,
    ],
)
def _push(bank_hbm, value_hbm, out_hbm, buf0, buf1, si0, si1, so0, so1):
    wid = lax.axis_index("s") * _NC + lax.axis_index("c")
    bufs, sin, sout = (buf0, buf1), (si0, si1), (so0, so1)
    out_handles = [None, None]
    for i in range(_NITER):
        b = i & 1
        if out_handles[b] is not None:
            out_handles[b].wait()
        c = jnp.minimum(wid + i * _NW, _NCHUNK - 1)
        row = jnp.minimum(c * _CH, K - _CH)
        row = pl.multiple_of(row, 8)

        @pl.when(c < _VCH)
        def _():
            pltpu.make_async_copy(
                value_hbm.at[pl.ds(row, _CH)], bufs[b], sin[b]).start()

        @pl.when(c >= _VCH)
        def _():
            pltpu.make_async_copy(
                bank_hbm.at[pl.ds(row, _CH)], bufs[b], sin[b]).start()

        pltpu.make_async_copy(
            bank_hbm.at[pl.ds(row, _CH)], bufs[b], sin[b]).wait()

        oh = pltpu.make_async_copy(bufs[b], out_hbm.at[pl.ds(row, _CH)], sout[b])
        oh.start()
        out_handles[b] = oh
    for oh in out_handles:
        oh.wait()


def kernel(bank, value):
    return _push(bank, value)


# speedup vs baseline: 1.0469x; 1.0469x over previous
import functools

import jax
import jax.numpy as jnp
from jax import lax
from jax.experimental import pallas as pl
from jax.experimental.pallas import tpu as pltpu
from jax.experimental.pallas import tpu_sc as plsc

K = 100000
DIM = 128
B = 16384

_INFO = plsc.get_sparse_core_info()
_NC, _NS = _INFO.num_cores, _INFO.num_subcores
_NW = _NC * _NS

_CH = 256
_VCH = B // _CH
_NCHUNK = _VCH + (K - B + _CH - 1) // _CH
_NITER = (_NCHUNK + _NW - 1) // _NW


@functools.partial(
    pl.kernel,
    mesh=plsc.VectorSubcoreMesh(core_axis_name="c", subcore_axis_name="s"),
    out_type=jax.ShapeDtypeStruct((K, DIM), jnp.float32),
    scratch_types=[
        pltpu.VMEM((_CH, DIM), jnp.float32),
        pltpu.VMEM((_CH, DIM), jnp.float32),
        pltpu.VMEM((_CH, DIM), jnp.float32),
        pltpu.SemaphoreType.DMA,
        pltpu.SemaphoreType.DMA,
        pltpu.SemaphoreType.DMA,
        pltpu.SemaphoreType.DMA,
        pltpu.SemaphoreType.DMA,
        pltpu.SemaphoreType.DMA,
    ],
)
def _push(bank_hbm, value_hbm, out_hbm, buf0, buf1, buf2,
          si0, si1, si2, so0, so1, so2):
    wid = lax.axis_index("s") * _NC + lax.axis_index("c")
    bufs, sin, sout = (buf0, buf1, buf2), (si0, si1, si2), (so0, so1, so2)

    def chunk_row(i):
        c = jnp.minimum(wid + i * _NW, _NCHUNK - 1)
        row = jnp.minimum(c * _CH, K - _CH)
        return c, pl.multiple_of(row, 8)

    def start_in(i):
        b = i % 3
        c, row = chunk_row(i)

        @pl.when(c < _VCH)
        def _():
            pltpu.make_async_copy(
                value_hbm.at[pl.ds(row, _CH)], bufs[b], sin[b]).start()

        @pl.when(c >= _VCH)
        def _():
            pltpu.make_async_copy(
                bank_hbm.at[pl.ds(row, _CH)], bufs[b], sin[b]).start()

    out_handles = [None, None, None]
    start_in(0)
    for i in range(_NITER):
        b = i % 3
        if i + 1 < _NITER:
            bn = (i + 1) % 3
            if out_handles[bn] is not None:
                out_handles[bn].wait()
                out_handles[bn] = None
            start_in(i + 1)
        _, row = chunk_row(i)
        pltpu.make_async_copy(
            bank_hbm.at[pl.ds(row, _CH)], bufs[b], sin[b]).wait()

        oh = pltpu.make_async_copy(bufs[b], out_hbm.at[pl.ds(row, _CH)], sout[b])
        oh.start()
        out_handles[b] = oh
    for oh in out_handles:
        if oh is not None:
            oh.wait()


def kernel(bank, value):
    return _push(bank, value)


# speedup vs baseline: 1.9672x; 1.8790x over previous
import jax
import jax.numpy as jnp
from jax.experimental import pallas as pl
from jax.experimental.pallas import tpu as pltpu

K = 100000
DIM = 128
B = 16384

_BR = 8192
_VAL_BLOCKS = B // _BR
_GRID = (K + _BR - 1) // _BR


def _push_body(bank_ref, value_ref, out_ref):
    i = pl.program_id(0)

    @pl.when(i < _VAL_BLOCKS)
    def _():
        out_ref[...] = value_ref[...]

    @pl.when(i >= _VAL_BLOCKS)
    def _():
        out_ref[...] = bank_ref[...]


@jax.jit
def kernel(bank, value):
    return pl.pallas_call(
        _push_body,
        grid=(_GRID,),
        in_specs=[
            pl.BlockSpec((_BR, DIM), lambda i: (jnp.maximum(i, _VAL_BLOCKS), 0)),
            pl.BlockSpec((_BR, DIM), lambda i: (jnp.minimum(i, _VAL_BLOCKS - 1), 0)),
        ],
        out_specs=pl.BlockSpec((_BR, DIM), lambda i: (i, 0)),
        out_shape=jax.ShapeDtypeStruct((K, DIM), jnp.float32),
    )(bank, value)


# speedup vs baseline: 2.0205x; 1.0271x over previous
import jax
import jax.numpy as jnp
from jax.experimental import pallas as pl
from jax.experimental.pallas import tpu as pltpu

K = 100000
DIM = 128
B = 16384

_CH = 4096
_NCH = (K + _CH - 1) // _CH
_VCH = B // _CH


def _rows(i):
    return min(_CH, K - i * _CH)


def _push_body(bank_ref, value_ref, out_ref, *scratch):
    bufs, sin, sout = scratch[:_NCH], scratch[_NCH:2 * _NCH], scratch[2 * _NCH:]
    ins, outs = [], []
    for i in range(_NCH):
        src = value_ref if i < _VCH else bank_ref
        n = _rows(i)
        ins.append(pltpu.make_async_copy(
            src.at[pl.ds(i * _CH, n)], bufs[i].at[pl.ds(0, n)], sin[i]))
        outs.append(pltpu.make_async_copy(
            bufs[i].at[pl.ds(0, n)], out_ref.at[pl.ds(i * _CH, n)], sout[i]))
    for c in ins:
        c.start()
    for i in range(_NCH):
        ins[i].wait()
        outs[i].start()
    for c in outs:
        c.wait()


@jax.jit
def kernel(bank, value):
    return pl.pallas_call(
        _push_body,
        out_shape=jax.ShapeDtypeStruct((K, DIM), jnp.float32),
        in_specs=[
            pl.BlockSpec(memory_space=pl.ANY),
            pl.BlockSpec(memory_space=pl.ANY),
        ],
        out_specs=pl.BlockSpec(memory_space=pl.ANY),
        scratch_shapes=(
            [pltpu.VMEM((_CH, DIM), jnp.float32)] * _NCH
            + [pltpu.SemaphoreType.DMA] * (2 * _NCH)
        ),
    )(bank, value)
